# baseline (speedup 1.0000x reference)
import jax
import jax.numpy as jnp
from jax.experimental import pallas as pl
from jax.experimental.pallas import tpu as pltpu

D = 1152
LE = 32
NT = 576
HID = 576
KB = 9
CB = D // KB


def _body(ht_ref, mem_ref, w1_ref, b1_ref, w2_ref, b2_ref, out_ref,
          hscr, hid_acc):
    j = pl.program_id(0)

    hj = jnp.mean(ht_ref[...], axis=0, keepdims=True)
    hscr[j, :, :] = hj
    part = jnp.dot(hj, w1_ref[...], preferred_element_type=jnp.float32)

    @pl.when(j == 0)
    def _init():
        hid_acc[...] = part

    @pl.when(j > 0)
    def _acc():
        hid_acc[...] += part

    @pl.when(j == KB - 1)
    def _finish():
        hid = jnp.maximum(hid_acc[...] + b1_ref[...], 0.0)
        logit = jnp.dot(hid, w2_ref[...],
                        preferred_element_type=jnp.float32) + b2_ref[...]
        commit = logit[0, 0] >= 0.0

        mem = mem_ref[...]
        hh = jnp.float32(0.0)
        mm = jnp.zeros((LE, 1), jnp.float32)
        mh = jnp.zeros((LE, 1), jnp.float32)
        for c in range(KB):
            hc = hscr[c, :, :]
            mc = mem[:, c * CB:(c + 1) * CB]
            hh = hh + jnp.sum(hc * hc)
            mm = mm + jnp.sum(mc * mc, axis=1, keepdims=True)
            mh = mh + jnp.sum(mc * hc, axis=1, keepdims=True)
        denom = (jnp.sqrt(mm) + 1e-8) * (jnp.sqrt(hh) + 1e-8)
        sims = mh / denom

        row_ids = jax.lax.broadcasted_iota(jnp.int32, sims.shape, 0)
        max_sim = jnp.max(sims)
        idx = jnp.min(jnp.where(sims == max_sim, row_ids,
                                jnp.iinfo(jnp.int32).max))
        mask = (row_ids == idx) & commit
        for c in range(KB):
            out_ref[:, c * CB:(c + 1) * CB] = jnp.where(
                mask, hscr[c, :, :], mem[:, c * CB:(c + 1) * CB])


def kernel(H_t, mem, W1, b1, W2, b2):
    return pl.pallas_call(
        _body,
        grid=(KB,),
        in_specs=[
            pl.BlockSpec((NT, CB), lambda j: (0, j)),
            pl.BlockSpec((LE, D), lambda j: (0, 0)),
            pl.BlockSpec((CB, HID), lambda j: (j, 0)),
            pl.BlockSpec((1, HID), lambda j: (0, 0)),
            pl.BlockSpec((HID, 1), lambda j: (0, 0)),
            pl.BlockSpec((1, 1), lambda j: (0, 0)),
        ],
        out_specs=pl.BlockSpec((LE, D), lambda j: (0, 0)),
        out_shape=jax.ShapeDtypeStruct(mem.shape, mem.dtype),
        scratch_shapes=[
            pltpu.VMEM((KB, 1, CB), jnp.float32),
            pltpu.VMEM((1, HID), jnp.float32),
        ],
    )(H_t, mem, W1, b1.reshape(1, -1), W2, b2.reshape(1, 1))
